# Initial kernel scaffold; baseline (speedup 1.0000x reference)
import jax
import jax.numpy as jnp
from jax.experimental import pallas as pl


def kernel(x, edge_index, W1, b1, W2, b2):
    raise NotImplementedError("write your pallas kernel here")


# speedup vs baseline: 15.0911x; 15.0911x over previous
import functools

import jax
import jax.numpy as jnp
from jax import lax
from jax.experimental import pallas as pl
from jax.experimental.pallas import tpu as pltpu
from jax.experimental.pallas import tpu_sc as plsc

N = 50000
E = 800000
D_IN = 64
D_HID = 128
D_OUT = 10

NC = 2
NS = 16
NW = NC * NS

SL = 3136
N_PAD = NS * SL
SINK = N + 8

CHUNK = 128
E_PAD = 802816
EPW = E_PAD // NW
NCH = EPW // CHUNK
EPS = E_PAD // NS
NCH1 = EPS // CHUNK

BR = 2000
GRID = N // BR

SLC = SL // 8


def _sc_mesh():
    return plsc.VectorSubcoreMesh(core_axis_name="c", subcore_axis_name="s")


@functools.partial(
    pl.kernel,
    out_type=jax.ShapeDtypeStruct((NC * N_PAD,), jnp.float32),
    mesh=_sc_mesh(),
    compiler_params=pltpu.CompilerParams(use_tc_tiling_on_sc=False),
    scratch_types=[
        pltpu.VMEM((CHUNK,), jnp.int32),
        pltpu.VMEM((CHUNK,), jnp.float32),
        pltpu.VMEM((SL,), jnp.float32),
        pltpu.VMEM_SHARED((N_PAD,), jnp.float32),
    ],
)
def _sc_hist(dst_hbm, z_hbm, out_hbm, didx, ones, buf, acc):
    c = lax.axis_index("c")
    s = lax.axis_index("s")
    row0 = pl.multiple_of(s * SL, 8)
    pltpu.sync_copy(z_hbm.at[pl.ds(row0, SL)], buf)
    pltpu.sync_copy(buf, acc.at[pl.ds(row0, SL)])
    for k in range(CHUNK // 16):
        ones[pl.ds(16 * k, 16)] = jnp.ones((16,), jnp.float32)
    plsc.subcore_barrier()
    base = (s * NC + c) * EPW

    def body(j, carry):
        off = pl.multiple_of(base + j * CHUNK, 8)
        pltpu.sync_copy(dst_hbm.at[pl.ds(off, CHUNK)], didx)
        pltpu.sync_copy(ones, acc.at[didx], add=True)
        return carry

    lax.fori_loop(0, NCH, body, 0)
    plsc.subcore_barrier()
    obase = pl.multiple_of(c * N_PAD + row0, 8)
    pltpu.sync_copy(acc.at[pl.ds(row0, SL)], buf)
    pltpu.sync_copy(buf, out_hbm.at[pl.ds(obase, SL)])


@functools.partial(
    pl.kernel,
    out_type=jax.ShapeDtypeStruct((NC, N_PAD, 32), jnp.float32),
    mesh=_sc_mesh(),
    compiler_params=pltpu.CompilerParams(use_tc_tiling_on_sc=False),
    scratch_types=[
        pltpu.VMEM((CHUNK,), jnp.int32),
        pltpu.VMEM((CHUNK,), jnp.int32),
        pltpu.VMEM((CHUNK, 32), jnp.float32),
        pltpu.VMEM((SLC, 32), jnp.float32),
        pltpu.VMEM_SHARED((N_PAD, 32), jnp.float32),
    ],
)
def _sc_agg1(xp0_hbm, xp1_hbm, src_hbm, dst_hbm, z_hbm, out_hbm,
             sidx, didx, rows, buf, acc):
    c = lax.axis_index("c")
    s = lax.axis_index("s")
    row0 = pl.multiple_of(s * SL, 8)
    for k in range(SL // SLC):
        r = pl.multiple_of(row0 + k * SLC, 8)
        pltpu.sync_copy(z_hbm.at[pl.ds(r, SLC)], buf)
        pltpu.sync_copy(buf, acc.at[pl.ds(r, SLC)])
    plsc.subcore_barrier()
    base = s * EPS

    def body(j, carry):
        off = pl.multiple_of(base + j * CHUNK, 8)
        pltpu.sync_copy(src_hbm.at[pl.ds(off, CHUNK)], sidx)
        pltpu.sync_copy(dst_hbm.at[pl.ds(off, CHUNK)], didx)

        @pl.when(c == 0)
        def _():
            pltpu.sync_copy(xp0_hbm.at[sidx], rows)

        @pl.when(c == 1)
        def _():
            pltpu.sync_copy(xp1_hbm.at[sidx], rows)

        pltpu.sync_copy(rows, acc.at[didx], add=True)
        return carry

    lax.fori_loop(0, NCH1, body, 0)
    plsc.subcore_barrier()
    for k in range(SL // SLC):
        r = pl.multiple_of(row0 + k * SLC, 8)
        pltpu.sync_copy(acc.at[pl.ds(r, SLC)], buf)
        pltpu.sync_copy(buf, out_hbm.at[c, pl.ds(r, SLC)])


@functools.partial(
    pl.kernel,
    out_type=jax.ShapeDtypeStruct((NC, N_PAD, 16), jnp.float32),
    mesh=_sc_mesh(),
    compiler_params=pltpu.CompilerParams(use_tc_tiling_on_sc=False),
    scratch_types=[
        pltpu.VMEM((CHUNK,), jnp.int32),
        pltpu.VMEM((CHUNK,), jnp.int32),
        pltpu.VMEM((CHUNK, 16), jnp.float32),
        pltpu.VMEM((SL, 16), jnp.float32),
        pltpu.VMEM_SHARED((N_PAD, 16), jnp.float32),
    ],
)
def _sc_agg2(tp_hbm, src_hbm, dst_hbm, z_hbm, out_hbm, sidx, didx, rows, buf,
             acc):
    c = lax.axis_index("c")
    s = lax.axis_index("s")
    row0 = pl.multiple_of(s * SL, 8)
    pltpu.sync_copy(z_hbm.at[pl.ds(row0, SL)], buf)
    pltpu.sync_copy(buf, acc.at[pl.ds(row0, SL)])
    plsc.subcore_barrier()
    base = (s * NC + c) * EPW

    def body(j, carry):
        off = pl.multiple_of(base + j * CHUNK, 8)
        pltpu.sync_copy(src_hbm.at[pl.ds(off, CHUNK)], sidx)
        pltpu.sync_copy(dst_hbm.at[pl.ds(off, CHUNK)], didx)
        pltpu.sync_copy(tp_hbm.at[sidx], rows)
        pltpu.sync_copy(rows, acc.at[didx], add=True)
        return carry

    lax.fori_loop(0, NCH, body, 0)
    plsc.subcore_barrier()
    pltpu.sync_copy(acc.at[pl.ds(row0, SL)], buf)
    pltpu.sync_copy(buf, out_hbm.at[c, pl.ds(row0, SL)])


def _tc_prep_body(degp_ref, x_ref, dinv_ref, xp0_ref, xp1_ref):
    deg = degp_ref[0] + degp_ref[1] + 1.0
    dinv = lax.rsqrt(deg)
    dinv_ref[...] = dinv
    xp = x_ref[...] * dinv
    xp0_ref[...] = xp[:, :32]
    xp1_ref[...] = xp[:, 32:]


def _tc_prep(degp, x):
    return pl.pallas_call(
        _tc_prep_body,
        grid=(GRID,),
        in_specs=[
            pl.BlockSpec((NC, BR, 1), lambda i: (0, i, 0)),
            pl.BlockSpec((BR, D_IN), lambda i: (i, 0)),
        ],
        out_specs=[
            pl.BlockSpec((BR, 1), lambda i: (i, 0)),
            pl.BlockSpec((BR, 32), lambda i: (i, 0)),
            pl.BlockSpec((BR, 32), lambda i: (i, 0)),
        ],
        out_shape=[
            jax.ShapeDtypeStruct((N, 1), jnp.float32),
            jax.ShapeDtypeStruct((N, 32), jnp.float32),
            jax.ShapeDtypeStruct((N, 32), jnp.float32),
        ],
    )(degp, x)


def _tc_mid_body(a1_ref, xp0_ref, xp1_ref, dinv_ref, W1_ref, b1_ref, W2_ref,
                 tp_ref):
    dinv = dinv_ref[...]
    y0 = dinv * (a1_ref[0] + xp0_ref[...])
    y1 = dinv * (a1_ref[1] + xp1_ref[...])
    y = jnp.concatenate([y0, y1], axis=1)
    h = jnp.dot(y, W1_ref[...], preferred_element_type=jnp.float32)
    h = jnp.maximum(h + b1_ref[...], 0.0)
    t = jnp.dot(h, W2_ref[...], preferred_element_type=jnp.float32)
    tp_ref[...] = dinv * t


def _tc_mid(a1, xp0, xp1, dinv, W1, b1r, W2p):
    return pl.pallas_call(
        _tc_mid_body,
        grid=(GRID,),
        in_specs=[
            pl.BlockSpec((NC, BR, 32), lambda i: (0, i, 0)),
            pl.BlockSpec((BR, 32), lambda i: (i, 0)),
            pl.BlockSpec((BR, 32), lambda i: (i, 0)),
            pl.BlockSpec((BR, 1), lambda i: (i, 0)),
            pl.BlockSpec((D_IN, D_HID), lambda i: (0, 0)),
            pl.BlockSpec((1, D_HID), lambda i: (0, 0)),
            pl.BlockSpec((D_HID, 16), lambda i: (0, 0)),
        ],
        out_specs=pl.BlockSpec((BR, 16), lambda i: (i, 0)),
        out_shape=jax.ShapeDtypeStruct((N, 16), jnp.float32),
    )(a1, xp0, xp1, dinv, W1, b1r, W2p)


def _tc_final_body(a2_ref, tp_ref, dinv_ref, b2_ref, sz_ref, sl_ref):
    i = pl.program_id(0)
    dinv = dinv_ref[...]
    z = dinv * (a2_ref[0] + a2_ref[1] + tp_ref[...]) + b2_ref[...]
    zz = z[:, :10]
    m = jnp.max(zz, axis=1, keepdims=True)
    lse = jnp.log(jnp.sum(jnp.exp(zz - m), axis=1, keepdims=True)) + m
    psz = jnp.sum(z, axis=0, keepdims=True)
    psl = jnp.sum(lse).reshape(1, 1)

    @pl.when(i == 0)
    def _():
        sz_ref[...] = psz
        sl_ref[...] = psl

    @pl.when(i != 0)
    def _():
        sz_ref[...] += psz
        sl_ref[...] += psl


def _tc_final(a2, tp, dinv, b2r):
    return pl.pallas_call(
        _tc_final_body,
        grid=(GRID,),
        in_specs=[
            pl.BlockSpec((NC, BR, 16), lambda i: (0, i, 0)),
            pl.BlockSpec((BR, 16), lambda i: (i, 0)),
            pl.BlockSpec((BR, 1), lambda i: (i, 0)),
            pl.BlockSpec((1, 16), lambda i: (0, 0)),
        ],
        out_specs=[
            pl.BlockSpec((1, 16), lambda i: (0, 0)),
            pl.BlockSpec((1, 1), lambda i: (0, 0)),
        ],
        out_shape=[
            jax.ShapeDtypeStruct((1, 16), jnp.float32),
            jax.ShapeDtypeStruct((1, 1), jnp.float32),
        ],
    )(a2, tp, dinv, b2r)


def kernel(x, edge_index, W1, b1, W2, b2):
    src = edge_index[0]
    dst = edge_index[1]
    pad_e = E_PAD - E
    src_p = jnp.concatenate([src, jnp.zeros((pad_e,), jnp.int32)])
    dst_p = jnp.concatenate([dst, jnp.full((pad_e,), SINK, jnp.int32)])
    zeros1 = jnp.zeros((N_PAD,), jnp.float32)
    zeros32 = jnp.zeros((N_PAD, 32), jnp.float32)
    zeros16 = jnp.zeros((N_PAD, 16), jnp.float32)
    b1r = b1.reshape(1, D_HID)
    W2p = jnp.pad(W2, ((0, 0), (0, 16 - D_OUT)))
    b2r = jnp.pad(b2, (0, 16 - D_OUT)).reshape(1, 16)

    degp = _sc_hist(dst_p, zeros1)
    dinv, xp0, xp1 = _tc_prep(degp.reshape(NC, N_PAD, 1), x)
    a1 = _sc_agg1(xp0, xp1, src_p, dst_p, zeros32)
    tp = _tc_mid(a1, xp0, xp1, dinv, W1, b1r, W2p)
    a2 = _sc_agg2(tp, src_p, dst_p, zeros16)
    sz, sl = _tc_final(a2, tp, dinv, b2r)
    out = (sz[0, :D_OUT] - sl[0, 0]) / float(N)
    return out.reshape(1, D_OUT)
